# baseline (speedup 1.0000x reference)
import jax
import jax.numpy as jnp
from jax.experimental import pallas as pl

_N = 512
_T = 64
_NA = 1
_W1 = _N - 1
_W2 = _T - _NA - 1
_TB = 8


def _body(x_ref, d1_ref, d2_ref):
    p = pl.program_id(0)
    xv = x_ref[...]
    valid = jnp.where(jnp.isnan(xv), 0.0, 1.0)
    rows = _TB * _N
    lane = jax.lax.broadcasted_iota(jnp.int32, (rows, _T), 1)
    trow = jax.lax.broadcasted_iota(jnp.int32, (rows, _T), 0) // _N + p * _TB
    vrep = jnp.concatenate([valid] * _TB, axis=0)
    vcol = jnp.sum(jnp.where(lane == trow, vrep, 0.0), axis=1, keepdims=True)
    row1 = jax.lax.broadcasted_iota(jnp.int32, (rows, _W1), 0) % _N
    col1 = jax.lax.broadcasted_iota(jnp.int32, (rows, _W1), 1)
    d1_ref[0] = jnp.where(row1 == col1 + 1, vcol, 0.0)
    col2 = jax.lax.broadcasted_iota(jnp.int32, (rows, _W2), 1)
    trow2 = jax.lax.broadcasted_iota(jnp.int32, (rows, _W2), 0) // _N + p * _TB
    d2_ref[0] = jnp.where(col2 == trow2 - (_NA + 1), vcol, 0.0)


def kernel(x):
    xt = jnp.transpose(x[0])
    d1, d2 = pl.pallas_call(
        _body,
        grid=(_T // _TB,),
        in_specs=[pl.BlockSpec((_N, _T), lambda p: (0, 0))],
        out_specs=[
            pl.BlockSpec((1, _TB * _N, _W1), lambda p: (0, p, 0)),
            pl.BlockSpec((1, _TB * _N, _W2), lambda p: (0, p, 0)),
        ],
        out_shape=[
            jax.ShapeDtypeStruct((1, _T * _N, _W1), jnp.float32),
            jax.ShapeDtypeStruct((1, _T * _N, _W2), jnp.float32),
        ],
    )(xt)
    return d1, d2


# speedup vs baseline: 1.7874x; 1.7874x over previous
import jax
import jax.numpy as jnp
from jax.experimental import pallas as pl

_N = 512
_T = 64
_NA = 1
_W1 = _N - 1
_W2 = _T - _NA - 1


def _body(x_ref, d1_ref, d2_ref):
    t = pl.program_id(0)
    xv = x_ref[...]
    valid = jnp.where(jnp.isnan(xv), 0.0, 1.0)
    lane = jax.lax.broadcasted_iota(jnp.int32, (_N, _T), 1)
    vcol = jnp.sum(jnp.where(lane == t, valid, 0.0), axis=1, keepdims=True)
    row = jax.lax.broadcasted_iota(jnp.int32, (_N, _W1), 0)
    col = jax.lax.broadcasted_iota(jnp.int32, (_N, _W1), 1)
    d1_ref[...] = jnp.where(row == col + 1, vcol, 0.0)
    col2 = jax.lax.broadcasted_iota(jnp.int32, (_N, _W2), 1)
    d2_ref[0] = jnp.where(col2 == t - (_NA + 1), vcol, 0.0)


def kernel(x):
    xt = jnp.transpose(x[0])
    d1, d2 = pl.pallas_call(
        _body,
        grid=(_T,),
        in_specs=[pl.BlockSpec((_N, _T), lambda t: (0, 0))],
        out_specs=[
            pl.BlockSpec((_N, _W1), lambda t: (t, 0)),
            pl.BlockSpec((1, _N, _W2), lambda t: (0, t, 0)),
        ],
        out_shape=[
            jax.ShapeDtypeStruct((_T * _N, _W1), jnp.float32),
            jax.ShapeDtypeStruct((1, _T * _N, _W2), jnp.float32),
        ],
    )(xt)
    return d1[None], d2
